# baseline (speedup 1.0000x reference)
import functools

import jax
import jax.numpy as jnp
from jax import lax
from jax.experimental import pallas as pl
from jax.experimental.pallas import tpu as pltpu
from jax.experimental.pallas import tpu_sc as plsc

_G = 200
_NBINS = _G * _G
_N = 25000 * 81
_NC = 2
_NS = 16
_NW = _NC * _NS
_CH = 7936
_KCH = 8
_PPW = _CH * _KCH
_CLIP_HI = _G - 1 - 1e-6
_NGRP = _CH // 16


def _sc_hist_body(pts_hbm, out_hbm, buf0, buf1, hist, sems):
  bufs = (buf0, buf1)
  c = lax.axis_index("c")
  s = lax.axis_index("s")
  wid = c * _NS + s
  base = wid * _PPW

  zeros16 = jnp.zeros((16,), jnp.float32)

  @pl.loop(0, _NBINS // 16, unroll=8)
  def _(i):
    hist[pl.ds(i * 16, 16)] = zeros16

  ones16 = jnp.ones((16,), jnp.float32)
  iota = lax.iota(jnp.int32, 16)
  iota2 = iota * 2
  iota2y = iota2 + 1

  def chunk_start(k):
    return jnp.minimum(base + k * _CH, _N - _CH)

  def start_dma(k, b):
    cs = chunk_start(k)
    return pltpu.async_copy(
        pts_hbm.at[pl.ds(cs * 2, _CH * 2)], bufs[b], sems.at[b]
    )

  start_dma(0, 0)
  start_dma(1, 1)

  def wait_dma(b):
    pltpu.make_async_copy(
        pts_hbm.at[pl.ds(0, _CH * 2)], bufs[b], sems.at[b]
    ).wait()

  def process_chunk(k, b):
    cs = chunk_start(k)
    wait_dma(b)
    d = (base + k * _CH) - cs

    @plsc.parallel_loop(0, _NGRP, unroll=8)
    def _(g):
      g32 = g * 32
      xv = plsc.load_gather(bufs[b], [iota2 + g32])
      yv = plsc.load_gather(bufs[b], [iota2y + g32])
      m = iota >= (d - g * 16)
      xc = jnp.clip(xv, 0.0, _CLIP_HI)
      yc = jnp.clip(yv, 0.0, _CLIP_HI)
      xi = (xc + 0.5).astype(jnp.int32)
      yi = (yc + 0.5).astype(jnp.int32)
      idx = xi * _G + yi
      plsc.addupdate_scatter(hist, [idx], ones16, mask=m)

    @pl.when(k + 2 < _KCH)
    def _():
      start_dma(k + 2, b)

  @pl.loop(0, _KCH, step=2)
  def _(k0):
    process_chunk(k0, 0)
    process_chunk(k0 + 1, 1)

  pltpu.sync_copy(hist, out_hbm.at[wid])


_sc_hist = pl.kernel(
    _sc_hist_body,
    out_type=jax.ShapeDtypeStruct((_NW, _NBINS), jnp.float32),
    mesh=plsc.VectorSubcoreMesh(core_axis_name="c", subcore_axis_name="s"),
    scratch_types=[
        pltpu.VMEM((_CH * 2,), jnp.float32),
        pltpu.VMEM((_CH * 2,), jnp.float32),
        pltpu.VMEM((_NBINS,), jnp.float32),
        pltpu.SemaphoreType.DMA((2,)),
    ],
    compiler_params=pltpu.CompilerParams(needs_layout_passes=False),
)


def _finalize_body(partials_ref, grid_ref, out_ref):
  total = jnp.sum(partials_ref[...], axis=0)
  prob = total / float(25000 * 80)
  out_ref[...] = jnp.where(grid_ref[...] != 0.0, 0.0, prob)


def kernel(all_points, grid):
  flat = all_points.reshape(_N * 2)
  partials = _sc_hist(flat)
  partials_3d = partials.reshape(_NW, _G, _G)
  out = pl.pallas_call(
      _finalize_body,
      out_shape=jax.ShapeDtypeStruct((_G, _G), jnp.float32),
  )(partials_3d, grid)
  return out


# speedup vs baseline: 12.6068x; 12.6068x over previous
import functools

import numpy as np
import jax
import jax.numpy as jnp
from jax import lax
from jax.experimental import pallas as pl
from jax.experimental.pallas import tpu as pltpu
from jax.experimental.pallas import tpu_sc as plsc

_G = 200
_NBINS = _G * _G
_N = 25000 * 81
_NPAD = 2 ** 21
_NC = 2
_NS = 16
_NW = _NC * _NS
_PPW = _NPAD // _NW
_CH = 8192
_KCH = _PPW // _CH
_NGRP = _CH // 16
_CLIP_HI = _G - 1 - 1e-6

_PAD_COUNTS = np.bincount(np.arange(_N, _NPAD) % 16, minlength=16).astype(
    np.float32
)
assert (_PAD_COUNTS[:8] == _PAD_COUNTS[0]).all()
assert (_PAD_COUNTS[8:] == _PAD_COUNTS[8]).all()


def _sc_hist_body(pts_hbm, out_hbm, xb0, yb0, xb1, yb1, hist, sems):
  xbufs = (xb0, xb1)
  ybufs = (yb0, yb1)
  c = lax.axis_index("c")
  s = lax.axis_index("s")
  wid = c * _NS + s
  base = wid * _PPW

  zeros16 = jnp.zeros((16,), jnp.float32)

  @pl.loop(0, _NBINS // 16, unroll=8)
  def _(i):
    hist[pl.ds(i * 16, 16)] = zeros16

  ones16 = jnp.ones((16,), jnp.float32)

  def start_dma(k, b):
    off = base + k * _CH
    pltpu.async_copy(pts_hbm.at[pl.ds(off, _CH)], xbufs[b], sems.at[b])
    pltpu.async_copy(
        pts_hbm.at[pl.ds(_NPAD + off, _CH)], ybufs[b], sems.at[b]
    )

  def wait_dma(b):
    pltpu.make_async_copy(
        pts_hbm.at[pl.ds(0, _CH)], xbufs[b], sems.at[b]
    ).wait()
    pltpu.make_async_copy(
        pts_hbm.at[pl.ds(0, _CH)], ybufs[b], sems.at[b]
    ).wait()

  start_dma(0, 0)
  start_dma(1, 1)

  def process_chunk(k, b):
    wait_dma(b)
    xbuf = xbufs[b]
    ybuf = ybufs[b]

    @plsc.parallel_loop(0, _NGRP, unroll=8)
    def _(g):
      g16 = g * 16
      xv = xbuf[pl.ds(g16, 16)]
      yv = ybuf[pl.ds(g16, 16)]
      xc = jnp.clip(xv, 0.0, _CLIP_HI)
      yc = jnp.clip(yv, 0.0, _CLIP_HI)
      xi = (xc + 0.5).astype(jnp.int32)
      yi = (yc + 0.5).astype(jnp.int32)
      idx = xi * _G + yi
      plsc.addupdate_scatter(hist, [idx], ones16)

    @pl.when(k + 2 < _KCH)
    def _():
      start_dma(k + 2, b)

  @pl.loop(0, _KCH, step=2)
  def _(k0):
    process_chunk(k0, 0)
    process_chunk(k0 + 1, 1)

  pltpu.sync_copy(hist, out_hbm.at[wid])


_sc_hist = pl.kernel(
    _sc_hist_body,
    out_type=jax.ShapeDtypeStruct((_NW, _NBINS), jnp.float32),
    mesh=plsc.VectorSubcoreMesh(core_axis_name="c", subcore_axis_name="s"),
    scratch_types=[
        pltpu.VMEM((_CH,), jnp.float32),
        pltpu.VMEM((_CH,), jnp.float32),
        pltpu.VMEM((_CH,), jnp.float32),
        pltpu.VMEM((_CH,), jnp.float32),
        pltpu.VMEM((_NBINS,), jnp.float32),
        pltpu.SemaphoreType.DMA((2,)),
    ],
    compiler_params=pltpu.CompilerParams(needs_layout_passes=False),
)


def _finalize_body(partials_ref, grid_ref, out_ref):
  total = jnp.sum(partials_ref[...], axis=0)
  rows = lax.broadcasted_iota(jnp.int32, (_G, _G), 0)
  cols = lax.broadcasted_iota(jnp.int32, (_G, _G), 1)
  pad_lo = float(_PAD_COUNTS[0])
  pad_hi = float(_PAD_COUNTS[8])
  pad_fix = jnp.where(
      (rows == 0) & (cols < 16),
      jnp.where(cols < 8, pad_lo, pad_hi),
      0.0,
  )
  total = total - pad_fix
  prob = total / float(25000 * 80)
  out_ref[...] = jnp.where(grid_ref[...] != 0.0, 0.0, prob)


def kernel(all_points, grid):
  pad_y = (jnp.arange(_N, _NPAD, dtype=jnp.int32) % 16).astype(jnp.float32)
  flat = jnp.zeros((2 * _NPAD,), jnp.float32)
  flat = flat.at[:_N].set(all_points[:, 0])
  flat = flat.at[_NPAD : _NPAD + _N].set(all_points[:, 1])
  flat = flat.at[_NPAD + _N :].set(pad_y)
  partials = _sc_hist(flat)
  partials_3d = partials.reshape(_NW, _G, _G)
  out = pl.pallas_call(
      _finalize_body,
      out_shape=jax.ShapeDtypeStruct((_G, _G), jnp.float32),
  )(partials_3d, grid)
  return out


# speedup vs baseline: 22.0094x; 1.7458x over previous
import functools

import numpy as np
import jax
import jax.numpy as jnp
from jax import lax
from jax.experimental import pallas as pl
from jax.experimental.pallas import tpu as pltpu
from jax.experimental.pallas import tpu_sc as plsc

_G = 200
_NBINS = _G * _G
_N = 25000 * 81
_NPAD = 2 ** 21
_NC = 2
_NS = 16
_NW = _NC * _NS
_PPW = _NPAD // _NW
_CH = 8192
_KCH = _PPW // _CH
_NGRP = _CH // 16
_CLIP_HI = _G - 1 - 1e-6

_PAD_COUNTS = np.bincount(np.arange(_N, _NPAD) % 16, minlength=16).astype(
    np.float32
)
assert (_PAD_COUNTS[:8] == _PAD_COUNTS[0]).all()
assert (_PAD_COUNTS[8:] == _PAD_COUNTS[8]).all()


def _sc_hist_body(xs_hbm, ys_hbm, out_hbm, xb0, yb0, xb1, yb1, hist, sems):
  xbufs = (xb0, xb1)
  ybufs = (yb0, yb1)
  c = lax.axis_index("c")
  s = lax.axis_index("s")
  wid = c * _NS + s
  base = wid * _PPW

  zeros16 = jnp.zeros((16,), jnp.float32)

  @pl.loop(0, _NBINS // 16, unroll=8)
  def _(i):
    hist[pl.ds(i * 16, 16)] = zeros16

  ones16 = jnp.ones((16,), jnp.float32)

  def start_dma(k, b):
    off = base + k * _CH
    pltpu.async_copy(xs_hbm.at[pl.ds(off, _CH)], xbufs[b], sems.at[b])
    pltpu.async_copy(ys_hbm.at[pl.ds(off, _CH)], ybufs[b], sems.at[b])

  def wait_dma(b):
    pltpu.make_async_copy(
        xs_hbm.at[pl.ds(0, _CH)], xbufs[b], sems.at[b]
    ).wait()
    pltpu.make_async_copy(
        ys_hbm.at[pl.ds(0, _CH)], ybufs[b], sems.at[b]
    ).wait()

  start_dma(0, 0)
  start_dma(1, 1)

  def process_chunk(k, b):
    wait_dma(b)
    xbuf = xbufs[b]
    ybuf = ybufs[b]

    @plsc.parallel_loop(0, _NGRP, unroll=8)
    def _(g):
      g16 = g * 16
      xv = xbuf[pl.ds(g16, 16)]
      yv = ybuf[pl.ds(g16, 16)]
      xc = jnp.clip(xv, 0.0, _CLIP_HI)
      yc = jnp.clip(yv, 0.0, _CLIP_HI)
      xi = (xc + 0.5).astype(jnp.int32)
      yi = (yc + 0.5).astype(jnp.int32)
      idx = xi * _G + yi
      plsc.addupdate_scatter(hist, [idx], ones16)

    @pl.when(k + 2 < _KCH)
    def _():
      start_dma(k + 2, b)

  @pl.loop(0, _KCH, step=2)
  def _(k0):
    process_chunk(k0, 0)
    process_chunk(k0 + 1, 1)

  pltpu.sync_copy(hist, out_hbm.at[wid])


_sc_hist = pl.kernel(
    _sc_hist_body,
    out_type=jax.ShapeDtypeStruct((_NW, _NBINS), jnp.float32),
    mesh=plsc.VectorSubcoreMesh(core_axis_name="c", subcore_axis_name="s"),
    scratch_types=[
        pltpu.VMEM((_CH,), jnp.float32),
        pltpu.VMEM((_CH,), jnp.float32),
        pltpu.VMEM((_CH,), jnp.float32),
        pltpu.VMEM((_CH,), jnp.float32),
        pltpu.VMEM((_NBINS,), jnp.float32),
        pltpu.SemaphoreType.DMA((2,)),
    ],
    compiler_params=pltpu.CompilerParams(needs_layout_passes=False),
)


def _finalize_body(partials_ref, grid_ref, out_ref):
  total = jnp.sum(partials_ref[...], axis=0)
  rows = lax.broadcasted_iota(jnp.int32, (_G, _G), 0)
  cols = lax.broadcasted_iota(jnp.int32, (_G, _G), 1)
  pad_lo = float(_PAD_COUNTS[0])
  pad_hi = float(_PAD_COUNTS[8])
  pad_fix = jnp.where(
      (rows == 0) & (cols < 16),
      jnp.where(cols < 8, pad_lo, pad_hi),
      0.0,
  )
  total = total - pad_fix
  prob = total / float(25000 * 80)
  out_ref[...] = jnp.where(grid_ref[...] != 0.0, 0.0, prob)


def kernel(all_points, grid):
  pad_y = (jnp.arange(_N, _NPAD, dtype=jnp.int32) % 16).astype(jnp.float32)
  pad_blk = jnp.stack([jnp.zeros((_NPAD - _N,), jnp.float32), pad_y])
  padded = jnp.concatenate([all_points.T, pad_blk], axis=1)
  partials = _sc_hist(padded[0], padded[1])
  partials_3d = partials.reshape(_NW, _G, _G)
  out = pl.pallas_call(
      _finalize_body,
      out_shape=jax.ShapeDtypeStruct((_G, _G), jnp.float32),
  )(partials_3d, grid)
  return out


# speedup vs baseline: 29.8057x; 1.3542x over previous
import functools

import jax
import jax.numpy as jnp
from jax import lax
from jax.experimental import pallas as pl
from jax.experimental.pallas import tpu as pltpu
from jax.experimental.pallas import tpu_sc as plsc

_G = 200
_NBINS = _G * _G
_N = 25000 * 81
_NPAD = 2 ** 21
_NC = 2
_NS = 16
_NW = _NC * _NS
_PPW = _NPAD // _NW
_CH = 8192
_KCH = _PPW // _CH
_NGRP = _CH // 16
_CLIP_HI = _G - 1 - 1e-6


def _sc_hist_body(xs_hbm, ys_hbm, out_hbm, xb0, yb0, xb1, yb1, hist, sems):
  xbufs = (xb0, xb1)
  ybufs = (yb0, yb1)
  c = lax.axis_index("c")
  s = lax.axis_index("s")
  wid = c * _NS + s
  base = wid * _PPW

  zeros16 = jnp.zeros((16,), jnp.float32)

  @pl.loop(0, _NBINS // 16, unroll=8)
  def _(i):
    hist[pl.ds(i * 16, 16)] = zeros16

  ones16 = jnp.ones((16,), jnp.float32)
  iota = lax.iota(jnp.int32, 16)

  def start_dma(k, b):
    off = base + k * _CH
    pltpu.async_copy(xs_hbm.at[pl.ds(off, _CH)], xbufs[b], sems.at[b])
    pltpu.async_copy(ys_hbm.at[pl.ds(off, _CH)], ybufs[b], sems.at[b])

  def wait_dma(b):
    pltpu.make_async_copy(
        xs_hbm.at[pl.ds(0, _CH)], xbufs[b], sems.at[b]
    ).wait()
    pltpu.make_async_copy(
        ys_hbm.at[pl.ds(0, _CH)], ybufs[b], sems.at[b]
    ).wait()

  start_dma(0, 0)
  start_dma(1, 1)

  def bin_index(xbuf, ybuf, g):
    g16 = g * 16
    xv = xbuf[pl.ds(g16, 16)]
    yv = ybuf[pl.ds(g16, 16)]
    xc = jnp.clip(xv, 0.0, _CLIP_HI)
    yc = jnp.clip(yv, 0.0, _CLIP_HI)
    xi = (xc + 0.5).astype(jnp.int32)
    yi = (yc + 0.5).astype(jnp.int32)
    return xi * _G + yi

  def process_chunk(k, b):
    wait_dma(b)
    xbuf = xbufs[b]
    ybuf = ybufs[b]
    thr = _N - (base + k * _CH)

    @pl.when(thr >= _CH)
    def _():
      @plsc.parallel_loop(0, _NGRP, unroll=8)
      def _(g):
        idx = bin_index(xbuf, ybuf, g)
        plsc.addupdate_scatter(hist, [idx], ones16)

    @pl.when(thr < _CH)
    def _():
      @plsc.parallel_loop(0, _NGRP, unroll=8)
      def _(g):
        idx = bin_index(xbuf, ybuf, g)
        m = (iota + g * 16) < thr
        plsc.addupdate_scatter(hist, [idx], ones16, mask=m)

    @pl.when(k + 2 < _KCH)
    def _():
      start_dma(k + 2, b)

  @pl.loop(0, _KCH, step=2)
  def _(k0):
    process_chunk(k0, 0)
    process_chunk(k0 + 1, 1)

  pltpu.sync_copy(hist, out_hbm.at[wid])


_sc_hist = pl.kernel(
    _sc_hist_body,
    out_type=jax.ShapeDtypeStruct((_NW, _NBINS), jnp.float32),
    mesh=plsc.VectorSubcoreMesh(core_axis_name="c", subcore_axis_name="s"),
    scratch_types=[
        pltpu.VMEM((_CH,), jnp.float32),
        pltpu.VMEM((_CH,), jnp.float32),
        pltpu.VMEM((_CH,), jnp.float32),
        pltpu.VMEM((_CH,), jnp.float32),
        pltpu.VMEM((_NBINS,), jnp.float32),
        pltpu.SemaphoreType.DMA((2,)),
    ],
    compiler_params=pltpu.CompilerParams(needs_layout_passes=False),
)


def _finalize_body(partials_ref, grid_ref, out_ref):
  total = jnp.sum(partials_ref[...], axis=0)
  prob = total / float(25000 * 80)
  out_ref[...] = jnp.where(grid_ref[...] != 0.0, 0.0, prob)


def kernel(all_points, grid):
  padded = jnp.zeros((2, _NPAD), jnp.float32).at[:, :_N].set(all_points.T)
  partials = _sc_hist(padded[0], padded[1])
  partials_3d = partials.reshape(_NW, _G, _G)
  out = pl.pallas_call(
      _finalize_body,
      out_shape=jax.ShapeDtypeStruct((_G, _G), jnp.float32),
  )(partials_3d, grid)
  return out
